# Initial kernel scaffold; baseline (speedup 1.0000x reference)
import jax
import jax.numpy as jnp
from jax.experimental import pallas as pl


def kernel(inputs, targets):
    raise NotImplementedError("write your pallas kernel here")


# speedup vs baseline: 4.7322x; 4.7322x over previous
import functools

import jax
import jax.numpy as jnp
from jax.experimental import pallas as pl
from jax.experimental.pallas import tpu as pltpu

K = 8192
HI = 64
LO = 128
BIN_W = 1.0 / K


def _hist_kernel(x_ref, t_ref, out_ref, h_ref, *, nb, nbatch):
    b = pl.program_id(0)
    i = pl.program_id(1)

    @pl.when((b == 0) & (i == 0))
    def _():
        h_ref[...] = jnp.zeros_like(h_ref)

    x = x_ref[0]
    labels = t_ref[0, 0]
    C, P = x.shape

    m = jnp.max(x, axis=0, keepdims=True)
    ex = jnp.exp(x - m)
    p = ex / jnp.sum(ex, axis=0, keepdims=True)

    cls = jax.lax.broadcasted_iota(jnp.int32, (C, P), 0)
    fg = cls == labels
    err = jnp.where(fg, 1.0 - p, p)
    bin_ = jnp.clip((err * K).astype(jnp.int32), 0, K - 1)
    rbin = (K - 1) - bin_
    hi = rbin >> 7
    lo = rbin & (LO - 1)
    akey = hi + fg.astype(jnp.int32) * HI

    iota_a = jax.lax.broadcasted_iota(jnp.int32, (C, P, 2 * HI), 2)
    iota_l = jax.lax.broadcasted_iota(jnp.int32, (C, P, LO), 2)
    A = (akey[:, :, None] == iota_a).astype(jnp.bfloat16)
    Bm = (lo[:, :, None] == iota_l).astype(jnp.bfloat16)
    h_ref[...] += jax.lax.dot_general(
        A, Bm, (((1,), (1,)), ((0,), (0,))),
        preferred_element_type=jnp.float32)

    @pl.when((b == nbatch - 1) & (i == nb - 1))
    def _():
        H = h_ref[...]
        bg_h = H[:, :HI, :]
        fg_h = H[:, HI:, :]

        io_r = jax.lax.broadcasted_iota(jnp.int32, (LO, LO), 0)
        io_c = jax.lax.broadcasted_iota(jnp.int32, (LO, LO), 1)
        upper = (io_r <= io_c).astype(jnp.float32)
        st = jnp.concatenate([bg_h, fg_h], axis=0)
        rowcum = jax.lax.dot_general(
            st, upper, (((2,), (0,)), ((), ())),
            preferred_element_type=jnp.float32)
        rowtot = rowcum[:, :, LO - 1]
        io_r2 = jax.lax.broadcasted_iota(jnp.int32, (HI, HI), 0)
        io_c2 = jax.lax.broadcasted_iota(jnp.int32, (HI, HI), 1)
        strict = (io_r2 < io_c2).astype(jnp.float32)
        exc = jax.lax.dot_general(
            rowtot, strict, (((1,), (0,)), ((), ())),
            preferred_element_type=jnp.float32)
        cum = rowcum + exc[:, :, None]

        b_cum = cum[:C]
        f_cum = cum[C:]
        g = f_cum[:, HI - 1, LO - 1]
        gb = g[:, None, None]
        jac = 1.0 - (gb - f_cum) / (gb + b_cum + 1e-6)
        sum_j = jnp.sum(jac, axis=(1, 2))
        j_last = jac[:, HI - 1, LO - 1]
        loss = BIN_W * sum_j - 0.5 * BIN_W * j_last
        present = (g > 0).astype(jnp.float32)
        num = jnp.sum(loss * present)
        den = jnp.maximum(jnp.sum(present), 1.0)
        out_ref[...] = (num / den).reshape(1, 1)


def kernel(inputs, targets):
    bt, c, h, w = inputs.shape
    npx = h * w
    p = 1024
    nb = npx // p
    x = inputs.reshape(bt, c, npx)
    t = targets.reshape(bt, nb, 1, p)
    out = pl.pallas_call(
        functools.partial(_hist_kernel, nb=nb, nbatch=bt),
        grid=(bt, nb),
        in_specs=[
            pl.BlockSpec((1, c, p), lambda b, i: (b, 0, i)),
            pl.BlockSpec((1, 1, 1, p), lambda b, i: (b, i, 0, 0)),
        ],
        out_specs=pl.BlockSpec((1, 1), lambda b, i: (0, 0)),
        out_shape=jax.ShapeDtypeStruct((1, 1), jnp.float32),
        scratch_shapes=[pltpu.VMEM((c, 2 * HI, LO), jnp.float32)],
    )(x, t)
    return out[0, 0]


# speedup vs baseline: 12.5213x; 2.6460x over previous
import functools

import jax
import jax.numpy as jnp
import numpy as np
from jax import lax
from jax.experimental import pallas as pl
from jax.experimental.pallas import tpu as pltpu
from jax.experimental.pallas import tpu_sc as plsc

K = 8192
HI = 64
LO = 128
BIN_W = 1.0 / K
NW = 32
CHUNK = 32768


def _bin_kernel(x_ref, t_ref, ids_ref):
    x = x_ref[0]
    labels = t_ref[0, 0]
    C, P = x.shape
    m = jnp.max(x, axis=0, keepdims=True)
    ex = jnp.exp(x - m)
    p = ex / jnp.sum(ex, axis=0, keepdims=True)
    cls = jax.lax.broadcasted_iota(jnp.int32, (C, P), 0)
    fg = cls == labels
    err = jnp.where(fg, 1.0 - p, p)
    bin_ = jnp.clip((err * K).astype(jnp.int32), 0, K - 1)
    rbin = (K - 1) - bin_
    ids_ref[...] = rbin + fg.astype(jnp.int32) * K


def _sc_hist_kernel(ids_hbm, out_hbm, table, buf, *, ntot, nhalf):
    wid = lax.axis_index("s") * 2 + lax.axis_index("c")
    base = jnp.where(wid < 26,
                     (wid // 2) * ntot + (wid % 2) * nhalf,
                     (wid - 13) * ntot)
    nchunks = jnp.where(wid < 26, nhalf // CHUNK, ntot // CHUNK)

    zeros16 = jnp.zeros((16,), jnp.float32)
    ones16 = jnp.ones((16,), jnp.float32)

    def zero_body(i, c):
        table[pl.ds(i * 16, 16)] = zeros16
        return c

    lax.fori_loop(0, (2 * K) // 16, zero_body, 0)

    def chunk_body(j, c):
        pltpu.sync_copy(ids_hbm.at[pl.ds(base + j * CHUNK, CHUNK)], buf)

        def scat_body(i, c2):
            idx = buf[pl.ds(i * 16, 16)]
            plsc.addupdate_scatter(table, [idx], ones16)
            return c2

        lax.fori_loop(0, CHUNK // 16, scat_body, 0)
        return c

    lax.fori_loop(0, nchunks, chunk_body, 0)
    pltpu.sync_copy(table, out_hbm.at[wid])


def _epilogue_kernel(h_ref, r_ref, out_ref):
    H = jax.lax.dot_general(
        r_ref[...], h_ref[...], (((1,), (0,)), ((), ())),
        preferred_element_type=jnp.float32)
    C = H.shape[0]
    bg_h = H[:, :HI, :]
    fg_h = H[:, HI:, :]
    io_r = jax.lax.broadcasted_iota(jnp.int32, (LO, LO), 0)
    io_c = jax.lax.broadcasted_iota(jnp.int32, (LO, LO), 1)
    upper = (io_r <= io_c).astype(jnp.float32)
    st = jnp.concatenate([bg_h, fg_h], axis=0)
    rowcum = jax.lax.dot_general(
        st, upper, (((2,), (0,)), ((), ())),
        preferred_element_type=jnp.float32)
    rowtot = rowcum[:, :, LO - 1]
    io_r2 = jax.lax.broadcasted_iota(jnp.int32, (HI, HI), 0)
    io_c2 = jax.lax.broadcasted_iota(jnp.int32, (HI, HI), 1)
    strict = (io_r2 < io_c2).astype(jnp.float32)
    exc = jax.lax.dot_general(
        rowtot, strict, (((1,), (0,)), ((), ())),
        preferred_element_type=jnp.float32)
    cum = rowcum + exc[:, :, None]
    b_cum = cum[:C]
    f_cum = cum[C:]
    g = f_cum[:, HI - 1, LO - 1]
    gb = g[:, None, None]
    jac = 1.0 - (gb - f_cum) / (gb + b_cum + 1e-6)
    sum_j = jnp.sum(jac, axis=(1, 2))
    j_last = jac[:, HI - 1, LO - 1]
    loss = BIN_W * sum_j - 0.5 * BIN_W * j_last
    present = (g > 0).astype(jnp.float32)
    num = jnp.sum(loss * present)
    den = jnp.maximum(jnp.sum(present), 1.0)
    out_ref[...] = (num / den).reshape(1, 1)


def kernel(inputs, targets):
    bt, c, h, w = inputs.shape
    npx = h * w
    p = 1024
    nb = npx // p
    ntot = bt * npx
    nhalf = ntot // 2
    x = inputs.reshape(bt, c, npx)
    t = targets.reshape(bt, nb, 1, p)

    ids = pl.pallas_call(
        _bin_kernel,
        grid=(bt, nb),
        in_specs=[
            pl.BlockSpec((1, c, p), lambda b, i: (b, 0, i)),
            pl.BlockSpec((1, 1, 1, p), lambda b, i: (b, i, 0, 0)),
        ],
        out_specs=pl.BlockSpec((c, p), lambda b, i: (0, b * nb + i)),
        out_shape=jax.ShapeDtypeStruct((c, ntot), jnp.int32),
    )(x, t)
    ids_flat = ids.reshape(c * ntot)

    mesh = plsc.VectorSubcoreMesh(core_axis_name="c", subcore_axis_name="s")
    sc_hist = pl.kernel(
        functools.partial(_sc_hist_kernel, ntot=ntot, nhalf=nhalf),
        mesh=mesh,
        out_type=jax.ShapeDtypeStruct((NW, 2 * K), jnp.float32),
        scratch_types=[
            pltpu.VMEM((2 * K,), jnp.float32),
            pltpu.VMEM((CHUNK,), jnp.int32),
        ],
        compiler_params=pltpu.CompilerParams(needs_layout_passes=False),
    )
    tables = sc_hist(ids_flat)

    red = np.zeros((c, NW), np.float32)
    for wid in range(NW):
        red[wid // 2 if wid < 26 else wid - 13, wid] = 1.0
    out = pl.pallas_call(
        _epilogue_kernel,
        in_specs=[
            pl.BlockSpec((NW, 2 * HI, LO), lambda: (0, 0, 0)),
            pl.BlockSpec((c, NW), lambda: (0, 0)),
        ],
        out_specs=pl.BlockSpec((1, 1), lambda: (0, 0)),
        out_shape=jax.ShapeDtypeStruct((1, 1), jnp.float32),
    )(tables.reshape(NW, 2 * HI, LO), jnp.asarray(red))
    return out[0, 0]


# speedup vs baseline: 40.6170x; 3.2438x over previous
import functools

import jax
import jax.numpy as jnp
import numpy as np
from jax import lax
from jax.experimental import pallas as pl
from jax.experimental.pallas import tpu as pltpu
from jax.experimental.pallas import tpu_sc as plsc

K = 8192
HI = 64
LO = 128
BIN_W = 1.0 / K
NW = 32
CHUNK = 32768
UNROLL = 8


def _bin_kernel(x_ref, t_ref, ids_ref):
    x = x_ref[0]
    labels = t_ref[0]
    m = jnp.max(x, axis=0, keepdims=True)
    ex = jnp.exp(x - m)
    p = ex / jnp.sum(ex, axis=0, keepdims=True)
    cls = jax.lax.broadcasted_iota(jnp.int32, x.shape, 0)
    fg = cls == labels[None]
    err = jnp.where(fg, 1.0 - p, p)
    bin_ = jnp.clip((err * K).astype(jnp.int32), 0, K - 1)
    rbin = (K - 1) - bin_
    ids_ref[...] = (rbin + fg.astype(jnp.int32) * K)[:, None]


def _sc_hist_kernel(ids_hbm, out_hbm, table, buf, *, ntot, nhalf):
    wid = lax.axis_index("s") * 2 + lax.axis_index("c")
    base = jnp.where(wid < 26,
                     (wid // 2) * ntot + (wid % 2) * nhalf,
                     (wid - 13) * ntot)
    nchunks = jnp.where(wid < 26, nhalf // CHUNK, ntot // CHUNK)

    zeros16 = jnp.zeros((16,), jnp.float32)
    ones16 = jnp.ones((16,), jnp.float32)

    def zero_body(i, c):
        table[pl.ds(i * 16, 16)] = zeros16
        return c

    lax.fori_loop(0, (2 * K) // 16, zero_body, 0)

    def chunk_body(j, c):
        pltpu.sync_copy(ids_hbm.at[pl.ds(base + j * CHUNK, CHUNK)], buf)

        def scat_body(i, c2):
            for u in range(UNROLL):
                idx = buf[pl.ds(i * (16 * UNROLL) + u * 16, 16)]
                plsc.addupdate_scatter(table, [idx], ones16)
            return c2

        lax.fori_loop(0, CHUNK // (16 * UNROLL), scat_body, 0)
        return c

    lax.fori_loop(0, nchunks, chunk_body, 0)
    pltpu.sync_copy(table, out_hbm.at[wid])


def _epilogue_kernel(h_ref, r_ref, out_ref):
    H = jax.lax.dot_general(
        r_ref[...], h_ref[...], (((1,), (0,)), ((), ())),
        preferred_element_type=jnp.float32)
    C = H.shape[0]
    bg_h = H[:, :HI, :]
    fg_h = H[:, HI:, :]
    io_r = jax.lax.broadcasted_iota(jnp.int32, (LO, LO), 0)
    io_c = jax.lax.broadcasted_iota(jnp.int32, (LO, LO), 1)
    upper = (io_r <= io_c).astype(jnp.float32)
    st = jnp.concatenate([bg_h, fg_h], axis=0)
    rowcum = jax.lax.dot_general(
        st, upper, (((2,), (0,)), ((), ())),
        preferred_element_type=jnp.float32)
    rowtot = rowcum[:, :, LO - 1]
    io_r2 = jax.lax.broadcasted_iota(jnp.int32, (HI, HI), 0)
    io_c2 = jax.lax.broadcasted_iota(jnp.int32, (HI, HI), 1)
    strict = (io_r2 < io_c2).astype(jnp.float32)
    exc = jax.lax.dot_general(
        rowtot, strict, (((1,), (0,)), ((), ())),
        preferred_element_type=jnp.float32)
    cum = rowcum + exc[:, :, None]
    b_cum = cum[:C]
    f_cum = cum[C:]
    g = f_cum[:, HI - 1, LO - 1]
    gb = g[:, None, None]
    jac = 1.0 - (gb - f_cum) / (gb + b_cum + 1e-6)
    sum_j = jnp.sum(jac, axis=(1, 2))
    j_last = jac[:, HI - 1, LO - 1]
    loss = BIN_W * sum_j - 0.5 * BIN_W * j_last
    present = (g > 0).astype(jnp.float32)
    num = jnp.sum(loss * present)
    den = jnp.maximum(jnp.sum(present), 1.0)
    out_ref[...] = (num / den).reshape(1, 1)


def kernel(inputs, targets):
    bt, c, h, w = inputs.shape
    rows = 8
    nrb = h // rows
    ntot = bt * h * w
    nhalf = ntot // 2

    ids = pl.pallas_call(
        _bin_kernel,
        grid=(bt, nrb),
        in_specs=[
            pl.BlockSpec((1, c, rows, w), lambda b, i: (b, 0, i, 0)),
            pl.BlockSpec((1, rows, w), lambda b, i: (b, i, 0)),
        ],
        out_specs=pl.BlockSpec((c, 1, rows, w), lambda b, i: (0, b, i, 0)),
        out_shape=jax.ShapeDtypeStruct((c, bt, h, w), jnp.int32),
    )(inputs, targets)
    ids_flat = ids.reshape(c * ntot)

    mesh = plsc.VectorSubcoreMesh(core_axis_name="c", subcore_axis_name="s")
    sc_hist = pl.kernel(
        functools.partial(_sc_hist_kernel, ntot=ntot, nhalf=nhalf),
        mesh=mesh,
        out_type=jax.ShapeDtypeStruct((NW, 2 * K), jnp.float32),
        scratch_types=[
            pltpu.VMEM((2 * K,), jnp.float32),
            pltpu.VMEM((CHUNK,), jnp.int32),
        ],
        compiler_params=pltpu.CompilerParams(needs_layout_passes=False),
    )
    tables = sc_hist(ids_flat)

    red = np.zeros((c, NW), np.float32)
    for wid in range(NW):
        red[wid // 2 if wid < 26 else wid - 13, wid] = 1.0
    out = pl.pallas_call(
        _epilogue_kernel,
        in_specs=[
            pl.BlockSpec((NW, 2 * HI, LO), lambda: (0, 0, 0)),
            pl.BlockSpec((c, NW), lambda: (0, 0)),
        ],
        out_specs=pl.BlockSpec((1, 1), lambda: (0, 0)),
        out_shape=jax.ShapeDtypeStruct((1, 1), jnp.float32),
    )(tables.reshape(NW, 2 * HI, LO), jnp.asarray(red))
    return out[0, 0]


# speedup vs baseline: 43.2319x; 1.0644x over previous
import functools

import jax
import jax.numpy as jnp
import numpy as np
from jax import lax
from jax.experimental import pallas as pl
from jax.experimental.pallas import tpu as pltpu
from jax.experimental.pallas import tpu_sc as plsc

K = 8192
HI = 64
LO = 128
BIN_W = 1.0 / K
NW = 32
CHUNK = 16384
UNROLL = 8


def _bin_kernel(x_ref, t_ref, ids_ref):
    x = x_ref[0]
    labels = t_ref[0]
    m = jnp.max(x, axis=0, keepdims=True)
    ex = jnp.exp(x - m)
    p = ex / jnp.sum(ex, axis=0, keepdims=True)
    cls = jax.lax.broadcasted_iota(jnp.int32, x.shape, 0)
    fg = cls == labels[None]
    err = jnp.where(fg, 1.0 - p, p)
    bin_ = jnp.clip((err * K).astype(jnp.int32), 0, K - 1)
    rbin = (K - 1) - bin_
    ids_ref[...] = (rbin + fg.astype(jnp.int32) * K)[:, None]


def _sc_hist_kernel(ids_hbm, out_hbm, table0, table1, buf0, buf1, sem0, sem1,
                    *, ntot, nhalf):
    wid = lax.axis_index("s") * 2 + lax.axis_index("c")
    base = jnp.where(wid < 26,
                     (wid // 2) * ntot + (wid % 2) * nhalf,
                     (wid - 13) * ntot)
    npairs = jnp.where(wid < 26, nhalf // (2 * CHUNK), ntot // (2 * CHUNK))

    zeros16 = jnp.zeros((16,), jnp.float32)
    ones16 = jnp.ones((16,), jnp.float32)

    def zero_body(i, c):
        table0[pl.ds(i * 16, 16)] = zeros16
        table1[pl.ds(i * 16, 16)] = zeros16
        return c

    lax.fori_loop(0, (2 * K) // 16, zero_body, 0)

    def copy_op(j, buf, sem):
        return pltpu.make_async_copy(
            ids_hbm.at[pl.ds(base + j * CHUNK, CHUNK)], buf, sem)

    def scatter(buf):
        def scat_body(i, c2):
            for u in range(UNROLL):
                idx = buf[pl.ds(i * (16 * UNROLL) + u * 16, 16)]
                plsc.addupdate_scatter(table0 if u % 2 == 0 else table1,
                                       [idx], ones16)
            return c2

        lax.fori_loop(0, CHUNK // (16 * UNROLL), scat_body, 0)

    copy_op(0, buf0, sem0).start()

    def pair_body(j2, c):
        j0 = 2 * j2
        copy_op(j0, buf0, sem0).wait()
        copy_op(j0 + 1, buf1, sem1).start()
        scatter(buf0)
        copy_op(j0 + 1, buf1, sem1).wait()

        @pl.when(j2 + 1 < npairs)
        def _():
            copy_op(j0 + 2, buf0, sem0).start()

        scatter(buf1)
        return c

    lax.fori_loop(0, npairs, pair_body, 0)
    pltpu.sync_copy(table0, out_hbm.at[wid, 0])
    pltpu.sync_copy(table1, out_hbm.at[wid, 1])


def _epilogue_kernel(h_ref, r_ref, out_ref):
    H = jax.lax.dot_general(
        r_ref[...], h_ref[...], (((1,), (0,)), ((), ())),
        preferred_element_type=jnp.float32)
    C = H.shape[0]
    bg_h = H[:, :HI, :]
    fg_h = H[:, HI:, :]
    io_r = jax.lax.broadcasted_iota(jnp.int32, (LO, LO), 0)
    io_c = jax.lax.broadcasted_iota(jnp.int32, (LO, LO), 1)
    upper = (io_r <= io_c).astype(jnp.float32)
    st = jnp.concatenate([bg_h, fg_h], axis=0)
    rowcum = jax.lax.dot_general(
        st, upper, (((2,), (0,)), ((), ())),
        preferred_element_type=jnp.float32)
    rowtot = rowcum[:, :, LO - 1]
    io_r2 = jax.lax.broadcasted_iota(jnp.int32, (HI, HI), 0)
    io_c2 = jax.lax.broadcasted_iota(jnp.int32, (HI, HI), 1)
    strict = (io_r2 < io_c2).astype(jnp.float32)
    exc = jax.lax.dot_general(
        rowtot, strict, (((1,), (0,)), ((), ())),
        preferred_element_type=jnp.float32)
    cum = rowcum + exc[:, :, None]
    b_cum = cum[:C]
    f_cum = cum[C:]
    g = f_cum[:, HI - 1, LO - 1]
    gb = g[:, None, None]
    jac = 1.0 - (gb - f_cum) / (gb + b_cum + 1e-6)
    sum_j = jnp.sum(jac, axis=(1, 2))
    j_last = jac[:, HI - 1, LO - 1]
    loss = BIN_W * sum_j - 0.5 * BIN_W * j_last
    present = (g > 0).astype(jnp.float32)
    num = jnp.sum(loss * present)
    den = jnp.maximum(jnp.sum(present), 1.0)
    out_ref[...] = (num / den).reshape(1, 1)


def kernel(inputs, targets):
    bt, c, h, w = inputs.shape
    rows = 8
    nrb = h // rows
    ntot = bt * h * w
    nhalf = ntot // 2

    ids = pl.pallas_call(
        _bin_kernel,
        grid=(bt, nrb),
        in_specs=[
            pl.BlockSpec((1, c, rows, w), lambda b, i: (b, 0, i, 0)),
            pl.BlockSpec((1, rows, w), lambda b, i: (b, i, 0)),
        ],
        out_specs=pl.BlockSpec((c, 1, rows, w), lambda b, i: (0, b, i, 0)),
        out_shape=jax.ShapeDtypeStruct((c, bt, h, w), jnp.int32),
    )(inputs, targets)
    ids_flat = ids.reshape(c * ntot)

    mesh = plsc.VectorSubcoreMesh(core_axis_name="c", subcore_axis_name="s")
    sc_hist = pl.kernel(
        functools.partial(_sc_hist_kernel, ntot=ntot, nhalf=nhalf),
        mesh=mesh,
        out_type=jax.ShapeDtypeStruct((NW, 2, 2 * K), jnp.float32),
        scratch_types=[
            pltpu.VMEM((2 * K,), jnp.float32),
            pltpu.VMEM((2 * K,), jnp.float32),
            pltpu.VMEM((CHUNK,), jnp.int32),
            pltpu.VMEM((CHUNK,), jnp.int32),
            pltpu.SemaphoreType.DMA,
            pltpu.SemaphoreType.DMA,
        ],
        compiler_params=pltpu.CompilerParams(needs_layout_passes=False),
    )
    tables = sc_hist(ids_flat)

    red = np.zeros((c, 2 * NW), np.float32)
    for wid in range(NW):
        cls = wid // 2 if wid < 26 else wid - 13
        red[cls, 2 * wid] = 1.0
        red[cls, 2 * wid + 1] = 1.0
    out = pl.pallas_call(
        _epilogue_kernel,
        in_specs=[
            pl.BlockSpec((2 * NW, 2 * HI, LO), lambda: (0, 0, 0)),
            pl.BlockSpec((c, 2 * NW), lambda: (0, 0)),
        ],
        out_specs=pl.BlockSpec((1, 1), lambda: (0, 0)),
        out_shape=jax.ShapeDtypeStruct((1, 1), jnp.float32),
    )(tables.reshape(2 * NW, 2 * HI, LO), jnp.asarray(red))
    return out[0, 0]


# speedup vs baseline: 63.9840x; 1.4800x over previous
import functools

import jax
import jax.numpy as jnp
import numpy as np
from jax import lax
from jax.experimental import pallas as pl
from jax.experimental.pallas import tpu as pltpu
from jax.experimental.pallas import tpu_sc as plsc

K = 8192
HI = 64
LO = 128
BIN_W = 1.0 / K
NW = 32
CHUNK = 8192
UNROLL = 8


def _bin_kernel(x_ref, t_ref, ids_ref):
    x = x_ref[0]
    labels = t_ref[0]
    m = jnp.max(x, axis=0, keepdims=True)
    ex = jnp.exp(x - m)
    p = ex / jnp.sum(ex, axis=0, keepdims=True)
    cls = jax.lax.broadcasted_iota(jnp.int32, x.shape, 0)
    fg = cls == labels[None]
    err = jnp.where(fg, 1.0 - p, p)
    bin_ = jnp.clip((err * K).astype(jnp.int32), 0, K - 1)
    rbin = (K - 1) - bin_
    ids = rbin + fg.astype(jnp.int32) * K
    r2 = ids.shape[1] // 2
    packed = ids[:, :r2] | (ids[:, r2:] << 16)
    ids_ref[...] = packed[:, None]


def _sc_hist_kernel(ids_hbm, out_hbm, table0, table1, buf0, buf1, sem0, sem1,
                    *, ntot, nhalf):
    wid = lax.axis_index("s") * 2 + lax.axis_index("c")
    base = jnp.where(wid < 26,
                     (wid // 2) * ntot + (wid % 2) * nhalf,
                     (wid - 13) * ntot)
    npairs = jnp.where(wid < 26, nhalf // (2 * CHUNK), ntot // (2 * CHUNK))

    zeros16 = jnp.zeros((16,), jnp.float32)
    ones16 = jnp.ones((16,), jnp.float32)

    def zero_body(i, c):
        table0[pl.ds(i * 16, 16)] = zeros16
        table1[pl.ds(i * 16, 16)] = zeros16
        return c

    lax.fori_loop(0, (2 * K) // 16, zero_body, 0)

    def copy_op(j, buf, sem):
        return pltpu.make_async_copy(
            ids_hbm.at[pl.ds(base + j * CHUNK, CHUNK)], buf, sem)

    def scatter(buf):
        def scat_body(i, c2):
            for u in range(UNROLL):
                v = buf[pl.ds(i * (16 * UNROLL) + u * 16, 16)]
                plsc.addupdate_scatter(table0, [v & 0xFFFF], ones16)
                plsc.addupdate_scatter(
                    table1, [lax.shift_right_logical(v, 16)], ones16)
            return c2

        lax.fori_loop(0, CHUNK // (16 * UNROLL), scat_body, 0)

    copy_op(0, buf0, sem0).start()

    def pair_body(j2, c):
        j0 = 2 * j2
        copy_op(j0, buf0, sem0).wait()
        copy_op(j0 + 1, buf1, sem1).start()
        scatter(buf0)
        copy_op(j0 + 1, buf1, sem1).wait()

        @pl.when(j2 + 1 < npairs)
        def _():
            copy_op(j0 + 2, buf0, sem0).start()

        scatter(buf1)
        return c

    lax.fori_loop(0, npairs, pair_body, 0)
    pltpu.sync_copy(table0, out_hbm.at[wid, 0])
    pltpu.sync_copy(table1, out_hbm.at[wid, 1])


def _epilogue_kernel(h_ref, r_ref, out_ref):
    H = jax.lax.dot_general(
        r_ref[...], h_ref[...], (((1,), (0,)), ((), ())),
        preferred_element_type=jnp.float32)
    C = H.shape[0]
    bg_h = H[:, :HI, :]
    fg_h = H[:, HI:, :]
    io_r = jax.lax.broadcasted_iota(jnp.int32, (LO, LO), 0)
    io_c = jax.lax.broadcasted_iota(jnp.int32, (LO, LO), 1)
    upper = (io_r <= io_c).astype(jnp.float32)
    st = jnp.concatenate([bg_h, fg_h], axis=0)
    rowcum = jax.lax.dot_general(
        st, upper, (((2,), (0,)), ((), ())),
        preferred_element_type=jnp.float32)
    rowtot = rowcum[:, :, LO - 1]
    io_r2 = jax.lax.broadcasted_iota(jnp.int32, (HI, HI), 0)
    io_c2 = jax.lax.broadcasted_iota(jnp.int32, (HI, HI), 1)
    strict = (io_r2 < io_c2).astype(jnp.float32)
    exc = jax.lax.dot_general(
        rowtot, strict, (((1,), (0,)), ((), ())),
        preferred_element_type=jnp.float32)
    cum = rowcum + exc[:, :, None]
    b_cum = cum[:C]
    f_cum = cum[C:]
    g = f_cum[:, HI - 1, LO - 1]
    gb = g[:, None, None]
    jac = 1.0 - (gb - f_cum) / (gb + b_cum + 1e-6)
    sum_j = jnp.sum(jac, axis=(1, 2))
    j_last = jac[:, HI - 1, LO - 1]
    loss = BIN_W * sum_j - 0.5 * BIN_W * j_last
    present = (g > 0).astype(jnp.float32)
    num = jnp.sum(loss * present)
    den = jnp.maximum(jnp.sum(present), 1.0)
    out_ref[...] = (num / den).reshape(1, 1)


def kernel(inputs, targets):
    bt, c, h, w = inputs.shape
    rows = 16
    nrb = h // rows
    ntot_w = bt * h * w // 2
    nhalf_w = ntot_w // 2

    ids = pl.pallas_call(
        _bin_kernel,
        grid=(bt, nrb),
        in_specs=[
            pl.BlockSpec((1, c, rows, w), lambda b, i: (b, 0, i, 0)),
            pl.BlockSpec((1, rows, w), lambda b, i: (b, i, 0)),
        ],
        out_specs=pl.BlockSpec((c, 1, rows // 2, w), lambda b, i: (0, b, i, 0)),
        out_shape=jax.ShapeDtypeStruct((c, bt, h // 2, w), jnp.int32),
    )(inputs, targets)
    ids_flat = ids.reshape(c * ntot_w)

    mesh = plsc.VectorSubcoreMesh(core_axis_name="c", subcore_axis_name="s")
    sc_hist = pl.kernel(
        functools.partial(_sc_hist_kernel, ntot=ntot_w, nhalf=nhalf_w),
        mesh=mesh,
        out_type=jax.ShapeDtypeStruct((NW, 2, 2 * K), jnp.float32),
        scratch_types=[
            pltpu.VMEM((2 * K,), jnp.float32),
            pltpu.VMEM((2 * K,), jnp.float32),
            pltpu.VMEM((CHUNK,), jnp.int32),
            pltpu.VMEM((CHUNK,), jnp.int32),
            pltpu.SemaphoreType.DMA,
            pltpu.SemaphoreType.DMA,
        ],
        compiler_params=pltpu.CompilerParams(needs_layout_passes=False),
    )
    tables = sc_hist(ids_flat)

    red = np.zeros((c, 2 * NW), np.float32)
    for wid in range(NW):
        cls = wid // 2 if wid < 26 else wid - 13
        red[cls, 2 * wid] = 1.0
        red[cls, 2 * wid + 1] = 1.0
    out = pl.pallas_call(
        _epilogue_kernel,
        in_specs=[
            pl.BlockSpec((2 * NW, 2 * HI, LO), lambda: (0, 0, 0)),
            pl.BlockSpec((c, 2 * NW), lambda: (0, 0)),
        ],
        out_specs=pl.BlockSpec((1, 1), lambda: (0, 0)),
        out_shape=jax.ShapeDtypeStruct((1, 1), jnp.float32),
    )(tables.reshape(2 * NW, 2 * HI, LO), jnp.asarray(red))
    return out[0, 0]
